# baseline (speedup 1.0000x reference)
import jax
import jax.numpy as jnp
from jax import lax
from jax.experimental import pallas as pl
from jax.experimental.pallas import tpu as pltpu
from jax.experimental.pallas import tpu_sc as plsc

NUM_USERS = 1000000
NUM_ITEMS = 1000000
EMBED_DIM = 32
BATCH = 16384

_info = plsc.get_sparse_core_info()
_NC, _NS, _L = _info.num_cores, _info.num_subcores, _info.num_lanes
_BPT = BATCH // _NS
_W = 16384
_NWIN = 61
_TAIL = NUM_USERS - _NWIN * _W
_WSUB = _W // _NS
_CAP = 2048
_PS = 16512
_DUMP = 17 * _PS


def _sc_body(uid_hbm, iid_hbm, uemb_hbm, iemb_hbm, ubias_hbm, ibias_hbm,
             tailu_hbm, taili_hbm, tailbu_hbm, tailbi_hbm,
             out_hbm,
             idrow_v, uoid_v, uopos_v, ioid_v, iopos_v, ubkt_v, ibkt_v,
             vslab_v, istage_v, fstage_v, tailu_v, taili_v, tailb_v,
             uv_v, iv_v, acc_v, spvals_u, spvals_i, segs_m, sem):
    c = lax.axis_index("c")
    sid = lax.axis_index("s")
    tbase = sid * _BPT
    lanes = lax.iota(jnp.int32, _L)

    for trj in range(2):
        tr = 2 * c + trj
        pltpu.sync_copy(tailu_hbm.at[tr], tailu_v.at[pl.ds(trj * 8, 8), :])
        pltpu.sync_copy(taili_hbm.at[tr], taili_v.at[pl.ds(trj * 8, 8), :])
    pltpu.sync_copy(tailbu_hbm, tailb_v.at[0])
    pltpu.sync_copy(tailbi_hbm, tailb_v.at[1])

    for ids_hbm, oid_v, opos_v in ((uid_hbm, uoid_v, uopos_v),
                                   (iid_hbm, ioid_v, iopos_v)):
        def blk(kb, cnt, ids_hbm=ids_hbm, oid_v=oid_v, opos_v=opos_v):
            pltpu.sync_copy(ids_hbm.at[pl.ds(kb * 128, 128)], idrow_v)
            for v in range(8):
                ids16 = idrow_v[pl.ds(v * _L, _L)]
                owner = (ids16 % _W) // _WSUB
                m = owner == sid
                cnt = jnp.minimum(cnt, _CAP - _L)
                plsc.store_compressed(oid_v.at[pl.ds(cnt, _L)], ids16,
                                      mask=m)
                plsc.store_compressed(
                    opos_v.at[pl.ds(cnt, _L)],
                    kb * 128 + v * _L + lanes, mask=m)
                cnt = cnt + plsc.all_reduce_population_count(m)[0]
            return cnt

        cnt = lax.fori_loop(0, BATCH // 128, blk, jnp.int32(0))
        if ids_hbm is uid_hbm:
            ucnt = cnt
        else:
            icnt = cnt

    for t, (oid_v, opos_v, bkt_v, n_own) in enumerate(
            ((uoid_v, uopos_v, ubkt_v, ucnt), (ioid_v, iopos_v, ibkt_v,
                                               icnt))):
        segs_m[t * 70] = jnp.int32(0)

        def bucket(w, bcnt, oid_v=oid_v, opos_v=opos_v, bkt_v=bkt_v,
                   n_own=n_own, t=t):
            def scan(k, bc):
                ids16 = oid_v[pl.ds(k * _L, _L)]
                pos16 = opos_v[pl.ds(k * _L, _L)]
                we = jnp.minimum(ids16 // _W, _NWIN)
                m = (we == w) & ((k * _L + lanes) < n_own)
                ll = ids16 - w * _W - sid * _WSUB
                pck = (ll << 14) | pos16
                bc = jnp.minimum(bc, _CAP - _L)
                plsc.store_compressed(bkt_v.at[pl.ds(bc, _L)], pck, mask=m)
                return bc + plsc.all_reduce_population_count(m)[0]

            bcnt = lax.fori_loop(0, (n_own + _L - 1) // _L, scan, bcnt)
            segs_m[t * 70 + w + 1] = bcnt
            return bcnt

        lax.fori_loop(0, _NWIN + 1, bucket, jnp.int32(0))

    def seg_loop(t, w, body):
        lo = segs_m[t * 70 + w]
        hi = segs_m[t * 70 + w + 1]

        def one(ch, carry):
            body(lo + ch * 128, hi)
            return carry

        lax.fori_loop(0, (hi - lo + 127) // 128, one, 0)

    def scatter_rows(nrows):
        cps = []
        for r in range(nrows):
            cps.append(pltpu.async_copy(
                fstage_v.at[r], spv_cur[0].at[istage_v.at[r]], sem))
        for cp in cps:
            cp.wait()

    spv_cur = [None]

    def window(w, carry):
        for t, (bkt_v, spv) in enumerate(((ubkt_v, spvals_u),
                                          (ibkt_v, spvals_i))):
            emb = uemb_hbm if t == 0 else iemb_hbm
            spv_cur[0] = spv
            for trj in range(2):
                tr = 2 * c + trj
                pltpu.sync_copy(
                    emb.at[tr, :, pl.ds(w * _W + sid * _WSUB, _WSUB)],
                    vslab_v)

                def gath(off, hi, bkt_v=bkt_v, trj=trj):
                    for v in range(8):
                        pck = bkt_v[pl.ds(off + v * _L, _L)]
                        ll = jnp.minimum(pck >> 14, _WSUB - 1)
                        pos = pck & 16383
                        valid = (off + v * _L + lanes) < hi
                        for r in range(8):
                            g = plsc.load_gather(
                                vslab_v,
                                [jnp.full((_L,), r, jnp.int32), ll])
                            fstage_v[r, pl.ds(v * _L, _L)] = g
                            tgt = (trj * 8 + r) * _PS + pos
                            tgt = jnp.where(valid, tgt, _DUMP + lanes)
                            istage_v[r, pl.ds(v * _L, _L)] = tgt
                    scatter_rows(8)

                seg_loop(t, w, gath)

            btab = ubias_hbm if t == 0 else ibias_hbm
            pltpu.sync_copy(
                btab.at[pl.ds(w * _W + sid * _WSUB, _WSUB)],
                vslab_v.at[0])

            def gbias(off, hi, bkt_v=bkt_v):
                for v in range(8):
                    pck = bkt_v[pl.ds(off + v * _L, _L)]
                    ll = jnp.minimum(pck >> 14, _WSUB - 1)
                    pos = pck & 16383
                    valid = (off + v * _L + lanes) < hi
                    g = plsc.load_gather(
                        vslab_v, [jnp.zeros((_L,), jnp.int32), ll])
                    fstage_v[0, pl.ds(v * _L, _L)] = g * jnp.float32(0.5)
                    tgt = 16 * _PS + pos
                    tgt = jnp.where(valid, tgt, _DUMP + lanes)
                    istage_v[0, pl.ds(v * _L, _L)] = tgt
                scatter_rows(1)

            seg_loop(t, w, gbias)
        return carry

    lax.fori_loop(0, _NWIN, window, 0)

    for t, (bkt_v, spv, tail_v) in enumerate(
            ((ubkt_v, spvals_u, tailu_v), (ibkt_v, spvals_i, taili_v))):
        spv_cur[0] = spv

        def gtail(off, hi, bkt_v=bkt_v, tail_v=tail_v, t=t):
            for rb in range(2):
                for v in range(8):
                    pck = bkt_v[pl.ds(off + v * _L, _L)]
                    ll = jnp.minimum(pck >> 14, _TAIL - 1)
                    pos = pck & 16383
                    valid = (off + v * _L + lanes) < hi
                    for r8 in range(8):
                        r = rb * 8 + r8
                        g = plsc.load_gather(
                            tail_v, [jnp.full((_L,), r, jnp.int32), ll])
                        fstage_v[r8, pl.ds(v * _L, _L)] = g
                        tgt = r * _PS + pos
                        tgt = jnp.where(valid, tgt, _DUMP + lanes)
                        istage_v[r8, pl.ds(v * _L, _L)] = tgt
                scatter_rows(8)
            for v in range(8):
                pck = bkt_v[pl.ds(off + v * _L, _L)]
                ll = jnp.minimum(pck >> 14, _TAIL - 1)
                pos = pck & 16383
                valid = (off + v * _L + lanes) < hi
                gb = plsc.load_gather(
                    tailb_v, [jnp.full((_L,), t, jnp.int32), ll])
                fstage_v[0, pl.ds(v * _L, _L)] = gb * jnp.float32(0.5)
                tgt = 16 * _PS + pos
                tgt = jnp.where(valid, tgt, _DUMP + lanes)
                istage_v[0, pl.ds(v * _L, _L)] = tgt
            scatter_rows(1)

        seg_loop(t, _NWIN, gtail)

    plsc.subcore_barrier()

    def zero(k, carry):
        acc_v[pl.ds(k * _L, _L)] = jnp.zeros((_L,), jnp.float32)
        return carry

    lax.fori_loop(0, _BPT // _L, zero, 0)

    def row(r, carry):
        pltpu.sync_copy(spvals_u.at[pl.ds(r * _PS + tbase, _BPT)], uv_v)
        pltpu.sync_copy(spvals_i.at[pl.ds(r * _PS + tbase, _BPT)], iv_v)

        def mac(k, carry2):
            s = pl.ds(k * _L, _L)
            acc_v[s] = acc_v[s] + uv_v[s] * iv_v[s]
            return carry2

        lax.fori_loop(0, _BPT // _L, mac, 0)
        return carry

    lax.fori_loop(0, 16, row, 0)

    pltpu.sync_copy(spvals_u.at[pl.ds(16 * _PS + tbase, _BPT)], uv_v)
    pltpu.sync_copy(spvals_i.at[pl.ds(16 * _PS + tbase, _BPT)], iv_v)

    def brow(k, carry):
        s = pl.ds(k * _L, _L)
        acc_v[s] = acc_v[s] + uv_v[s] + iv_v[s]
        return carry

    lax.fori_loop(0, _BPT // _L, brow, 0)

    pltpu.sync_copy(acc_v, out_hbm.at[pl.ds(c * BATCH + tbase, _BPT)])


def _add_body(a_ref, o_ref):
    o_ref[...] = a_ref[pl.ds(0, BATCH)] + a_ref[pl.ds(BATCH, BATCH)]


def kernel(user_ids, item_ids, user_emb, item_emb, user_bias_table,
           item_bias_table):
    uid = user_ids.astype(jnp.int32)
    iid = item_ids.astype(jnp.int32)
    uembT = user_emb.T.reshape(4, 8, NUM_USERS)
    iembT = item_emb.T.reshape(4, 8, NUM_ITEMS)
    ubias_flat = user_bias_table.reshape(NUM_USERS)
    ibias_flat = item_bias_table.reshape(NUM_ITEMS)
    tailu = uembT[:, :, _NWIN * _W:]
    taili = iembT[:, :, _NWIN * _W:]
    tailbu = ubias_flat[_NWIN * _W:]
    tailbi = ibias_flat[_NWIN * _W:]

    mesh = plsc.VectorSubcoreMesh(core_axis_name="c", subcore_axis_name="s")
    f = pl.kernel(
        _sc_body, mesh=mesh,
        out_type=jax.ShapeDtypeStruct((2 * BATCH,), jnp.float32),
        scratch_types=[
            pltpu.VMEM((128,), jnp.int32),
            pltpu.VMEM((_CAP,), jnp.int32),
            pltpu.VMEM((_CAP,), jnp.int32),
            pltpu.VMEM((_CAP,), jnp.int32),
            pltpu.VMEM((_CAP,), jnp.int32),
            pltpu.VMEM((_CAP + 128,), jnp.int32),
            pltpu.VMEM((_CAP + 128,), jnp.int32),
            pltpu.VMEM((8, _WSUB), jnp.float32),
            pltpu.VMEM((8, 128), jnp.int32),
            pltpu.VMEM((8, 128), jnp.float32),
            pltpu.VMEM((16, _TAIL), jnp.float32),
            pltpu.VMEM((16, _TAIL), jnp.float32),
            pltpu.VMEM((2, _TAIL), jnp.float32),
            pltpu.VMEM((_BPT,), jnp.float32),
            pltpu.VMEM((_BPT,), jnp.float32),
            pltpu.VMEM((_BPT,), jnp.float32),
            pltpu.VMEM_SHARED((_DUMP + 128,), jnp.float32),
            pltpu.VMEM_SHARED((_DUMP + 128,), jnp.float32),
            pltpu.SMEM((140,), jnp.int32),
            pltpu.SemaphoreType.DMA,
        ],
        compiler_params=pltpu.CompilerParams(
            needs_layout_passes=False, use_tc_tiling_on_sc=True),
    )
    p = f(uid, iid, uembT, iembT, ubias_flat, ibias_flat,
          tailu, taili, tailbu, tailbi)

    add = pl.pallas_call(
        _add_body,
        out_shape=jax.ShapeDtypeStruct((BATCH,), jnp.float32),
    )
    return add(p)


# speedup vs baseline: 1.0767x; 1.0767x over previous
import jax
import jax.numpy as jnp
from jax import lax
from jax.experimental import pallas as pl
from jax.experimental.pallas import tpu as pltpu
from jax.experimental.pallas import tpu_sc as plsc

NUM_USERS = 1000000
NUM_ITEMS = 1000000
EMBED_DIM = 32
BATCH = 16384

_info = plsc.get_sparse_core_info()
_NC, _NS, _L = _info.num_cores, _info.num_subcores, _info.num_lanes
_NW = _NC * _NS
_BPW = BATCH // _NW
_CHUNK = 128
_NCHUNK = _BPW // _CHUNK


def _body(uid_hbm, iid_hbm, uemb_hbm, iemb_hbm, ubias_hbm, ibias_hbm,
          out_hbm,
          uidx_v, iidx_v, urows_v, irows_v, ub_v, ib_v, out_v, sem):
    wid = lax.axis_index("c") * _NS + lax.axis_index("s")
    base = wid * _BPW
    crow = wid * _NCHUNK

    pltpu.sync_copy(uid_hbm.at[pl.ds(crow, _NCHUNK)], uidx_v)
    pltpu.sync_copy(iid_hbm.at[pl.ds(crow, _NCHUNK)], iidx_v)

    copies = []
    for j in range(_NCHUNK):
        copies.append(pltpu.async_copy(
            uemb_hbm.at[uidx_v.at[j]], urows_v.at[pl.ds(j * _CHUNK, _CHUNK), :],
            sem))
        copies.append(pltpu.async_copy(
            iemb_hbm.at[iidx_v.at[j]], irows_v.at[pl.ds(j * _CHUNK, _CHUNK), :],
            sem))
        copies.append(pltpu.async_copy(
            ubias_hbm.at[uidx_v.at[j]], ub_v.at[pl.ds(j * _CHUNK, _CHUNK)],
            sem))
        copies.append(pltpu.async_copy(
            ibias_hbm.at[iidx_v.at[j]], ib_v.at[pl.ds(j * _CHUNK, _CHUNK)],
            sem))
    for c in copies:
        c.wait()

    lanes = lax.iota(jnp.int32, _L)

    def group(t, carry):
        res = jnp.zeros((_L,), jnp.float32)
        for m in range(_L):
            r = t * _L + m
            u0 = urows_v[r, pl.ds(0, _L)]
            u1 = urows_v[r, pl.ds(_L, _L)]
            i0 = irows_v[r, pl.ds(0, _L)]
            i1 = irows_v[r, pl.ds(_L, _L)]
            s = jnp.sum(u0 * i0 + u1 * i1)
            res = jnp.where(lanes == m, s, res)
        res = res + ub_v[pl.ds(t * _L, _L)] + ib_v[pl.ds(t * _L, _L)]
        out_v[pl.ds(t * _L, _L)] = res
        return carry

    lax.fori_loop(0, _BPW // _L, group, 0)

    pltpu.sync_copy(out_v, out_hbm.at[pl.ds(base, _BPW)])


def kernel(user_ids, item_ids, user_emb, item_emb, user_bias_table,
           item_bias_table):
    uid2 = user_ids.astype(jnp.int32).reshape(_NW * _NCHUNK, _CHUNK)
    iid2 = item_ids.astype(jnp.int32).reshape(_NW * _NCHUNK, _CHUNK)
    ubias_flat = user_bias_table.reshape(NUM_USERS)
    ibias_flat = item_bias_table.reshape(NUM_ITEMS)

    mesh = plsc.VectorSubcoreMesh(core_axis_name="c", subcore_axis_name="s")
    f = pl.kernel(
        _body, mesh=mesh,
        out_type=jax.ShapeDtypeStruct((BATCH,), jnp.float32),
        scratch_types=[
            pltpu.VMEM((_NCHUNK, _CHUNK), jnp.int32),
            pltpu.VMEM((_NCHUNK, _CHUNK), jnp.int32),
            pltpu.VMEM((_BPW, EMBED_DIM), jnp.float32),
            pltpu.VMEM((_BPW, EMBED_DIM), jnp.float32),
            pltpu.VMEM((_BPW,), jnp.float32),
            pltpu.VMEM((_BPW,), jnp.float32),
            pltpu.VMEM((_BPW,), jnp.float32),
            pltpu.SemaphoreType.DMA,
        ],
        compiler_params=pltpu.CompilerParams(
            needs_layout_passes=False, use_tc_tiling_on_sc=False),
    )
    return f(uid2, iid2, user_emb, item_emb, ubias_flat, ibias_flat)
